# baseline (speedup 1.0000x reference)
import jax
import jax.numpy as jnp
from jax import lax
from jax.experimental import pallas as pl
from jax.experimental.pallas import tpu as pltpu

_C = 512
_V1 = 512
_V2 = 64


def _celu(x):
    return jnp.where(x > 0, x, jnp.exp(jnp.minimum(x, 0.0)) - 1.0)


def _dot00(a, b):
    return lax.dot_general(a, b, (((0,), (0,)), ((), ())),
                           preferred_element_type=jnp.float32, precision=lax.Precision.HIGHEST)


def _body(starts_ref, ends_ref,
          pos_ref, rgb_ref, gidx_ref, startv_ref,
          c1w1a_ref, c1w1b_ref, c1b1_ref, c1w2_ref, c1b2_ref,
          c2w1a_ref, c2w1b_ref, c2b1_ref, c2w2_ref, c2b2_ref,
          c3w1a_ref, c3w1b_ref, c3b1_ref, c3w2_ref, c3b2_ref,
          linw_ref, linb_ref,
          out_ref,
          pos_s, rgb_s, gidx_s, sem):
    g = pl.program_id(0)
    s = starts_ref[g]
    e = ends_ref[g]
    astart = (s // _C) * _C
    n_iter = (e - astart + _C - 1) // _C
    startv = startv_ref[0:1, :]
    iota1 = lax.broadcasted_iota(jnp.int32, (_C, _V1), 1)

    def chunk(j, want_rgb):
        off = astart + j * _C
        pltpu.make_async_copy(pos_ref.at[pl.ds(off, _C), :], pos_s, sem).start()
        pltpu.make_async_copy(pos_ref.at[pl.ds(off, _C), :], pos_s, sem).wait()
        pltpu.make_async_copy(gidx_ref.at[pl.ds(off, _C), :], gidx_s, sem).start()
        pltpu.make_async_copy(gidx_ref.at[pl.ds(off, _C), :], gidx_s, sem).wait()
        if want_rgb:
            pltpu.make_async_copy(rgb_ref.at[pl.ds(off, _C), :], rgb_s, sem).start()
            pltpu.make_async_copy(rgb_ref.at[pl.ds(off, _C), :], rgb_s, sem).wait()
        p = pos_s[...]
        gi = gidx_s[...]
        vox = jnp.floor((p - startv) * 4.0).astype(jnp.int32)
        k = (vox[:, 0:1] * 8 + vox[:, 1:2]) * 8 + vox[:, 2:3]
        key = jnp.where(gi == g, k, -1)
        oh = (key == iota1).astype(jnp.float32)
        return p, oh

    def pass_a(j, acc):
        psum, cnt = acc
        p, oh = chunk(j, False)
        psum = psum + _dot00(oh, p)
        cnt = cnt + _dot00(oh, jnp.ones((_C, 1), jnp.float32))
        return psum, cnt

    psum0 = jnp.zeros((_V1, 3), jnp.float32)
    cnt0 = jnp.zeros((_V1, 1), jnp.float32)
    psum, count1 = lax.fori_loop(0, n_iter, pass_a, (psum0, cnt0))
    pos1 = psum / jnp.maximum(count1, 1.0)

    def pass_b(j, agg):
        p, oh = chunk(j, True)
        r = rgb_s[...]
        rel = p - lax.dot_general(oh, pos1, (((1,), (0,)), ((), ())),
                                  preferred_element_type=jnp.float32, precision=lax.Precision.HIGHEST)
        h = jnp.maximum(
            jnp.dot(r, c1w1a_ref[...], preferred_element_type=jnp.float32, precision=lax.Precision.HIGHEST)
            + jnp.dot(rel, c1w1b_ref[...], preferred_element_type=jnp.float32, precision=lax.Precision.HIGHEST)
            + c1b1_ref[...], 0.0)
        rows = [jnp.max(oh * h[:, f:f + 1], axis=0, keepdims=True)
                for f in range(16)]
        return jnp.maximum(agg, jnp.concatenate(rows, axis=0))

    agg1 = lax.fori_loop(0, n_iter, pass_b, jnp.zeros((16, _V1), jnp.float32))
    f1 = _celu(_dot00(agg1, c1w2_ref[...]) + c1b2_ref[...])

    vox1 = jnp.floor((pos1 - startv) * 2.0).astype(jnp.int32)
    k1 = (vox1[:, 0:1] * 4 + vox1[:, 1:2]) * 4 + vox1[:, 2:3]
    key1 = jnp.where(count1 > 0, k1, -1)
    iota2 = lax.broadcasted_iota(jnp.int32, (_V1, _V2), 1)
    oh2 = (key1 == iota2).astype(jnp.float32)
    psum2 = _dot00(oh2, pos1)
    count2 = _dot00(oh2, jnp.ones((_V1, 1), jnp.float32))
    pos2 = psum2 / jnp.maximum(count2, 1.0)
    rel2 = pos1 - lax.dot_general(oh2, pos2, (((1,), (0,)), ((), ())),
                                  preferred_element_type=jnp.float32, precision=lax.Precision.HIGHEST)
    h2 = jnp.maximum(
        jnp.dot(f1, c2w1a_ref[...], preferred_element_type=jnp.float32, precision=lax.Precision.HIGHEST)
        + jnp.dot(rel2, c2w1b_ref[...], preferred_element_type=jnp.float32, precision=lax.Precision.HIGHEST)
        + c2b1_ref[...], 0.0)
    rows2 = [jnp.max(oh2 * h2[:, f:f + 1], axis=0, keepdims=True)
             for f in range(64)]
    agg2 = jnp.concatenate(rows2, axis=0)
    f2 = _celu(_dot00(agg2, c2w2_ref[...]) + c2b2_ref[...])

    h3 = jnp.maximum(
        jnp.dot(f2, c3w1a_ref[...], preferred_element_type=jnp.float32, precision=lax.Precision.HIGHEST)
        + jnp.dot(pos2, c3w1b_ref[...], preferred_element_type=jnp.float32, precision=lax.Precision.HIGHEST)
        + c3b1_ref[...], 0.0)
    h3 = h3 * (count2 > 0).astype(jnp.float32)
    agg3 = jnp.max(h3, axis=0, keepdims=True)
    f3 = _celu(jnp.dot(agg3, c3w2_ref[...], preferred_element_type=jnp.float32, precision=lax.Precision.HIGHEST)
               + c3b2_ref[...])
    res = (jnp.dot(f3, linw_ref[...], preferred_element_type=jnp.float32, precision=lax.Precision.HIGHEST)
           + linb_ref[...])
    out_ref[...] = res.reshape(1, 1, 256)


def kernel(rgb, pos, glimpse_member__glimpse_index, glimpse__center,
           glimpse__batch,
           c1_w1, c1_b1, c1_w2, c1_b2,
           c2_w1, c2_b1, c2_w2, c2_b2,
           c3_w1, c3_b1, c3_w2, c3_b2,
           lin_w, lin_b):
    del glimpse__batch
    num_g = glimpse__center.shape[0]
    gidx = glimpse_member__glimpse_index.astype(jnp.int32)
    noise = jax.random.uniform(jax.random.key(1), (3,), jnp.float32)
    startv = (jnp.min(pos, axis=0) - noise).reshape(1, 3)
    gr = jnp.arange(num_g, dtype=jnp.int32)
    starts = jnp.searchsorted(gidx, gr, side="left").astype(jnp.int32)
    ends = jnp.searchsorted(gidx, gr + 1, side="left").astype(jnp.int32)
    pos_p = jnp.pad(pos, ((0, _C), (0, 0)))
    rgb_p = jnp.pad(rgb, ((0, _C), (0, 0)))
    gidx_p = jnp.pad(gidx.reshape(-1, 1), ((0, _C), (0, 0)),
                     constant_values=num_g)

    full = lambda a: pl.BlockSpec(a.shape, lambda g, s_, e_: (0,) * a.ndim)
    hbm = pl.BlockSpec(memory_space=pl.ANY)
    operands = [pos_p, rgb_p, gidx_p, startv,
                c1_w1[:1], c1_w1[1:], c1_b1.reshape(1, -1), c1_w2,
                c1_b2.reshape(1, -1),
                c2_w1[:32], c2_w1[32:], c2_b1.reshape(1, -1), c2_w2,
                c2_b2.reshape(1, -1),
                c3_w1[:128], c3_w1[128:], c3_b1.reshape(1, -1), c3_w2,
                c3_b2.reshape(1, -1),
                lin_w, lin_b.reshape(1, -1)]
    grid_spec = pltpu.PrefetchScalarGridSpec(
        num_scalar_prefetch=2,
        grid=(num_g,),
        in_specs=[hbm, hbm, hbm] + [full(a) for a in operands[3:]],
        out_specs=pl.BlockSpec((1, 1, 256), lambda g, s_, e_: (g, 0, 0)),
        scratch_shapes=[
            pltpu.VMEM((_C, 3), jnp.float32),
            pltpu.VMEM((_C, 1), jnp.float32),
            pltpu.VMEM((_C, 1), jnp.int32),
            pltpu.SemaphoreType.DMA,
        ],
    )
    out = pl.pallas_call(
        _body,
        grid_spec=grid_spec,
        out_shape=jax.ShapeDtypeStruct((num_g, 1, 256), jnp.float32),
    )(starts, ends, *operands)
    return out.reshape(num_g, 256)


# speedup vs baseline: 1.1961x; 1.1961x over previous
import jax
import jax.numpy as jnp
from jax import lax
from jax.experimental import pallas as pl
from jax.experimental.pallas import tpu as pltpu

_C = 512
_V1 = 512
_V2 = 64


def _celu(x):
    return jnp.where(x > 0, x, jnp.exp(jnp.minimum(x, 0.0)) - 1.0)


def _dot00(a, b):
    return lax.dot_general(a, b, (((0,), (0,)), ((), ())),
                           preferred_element_type=jnp.float32, precision=lax.Precision.HIGHEST)


def _body(starts_ref, ends_ref,
          pts_ref, gidx_ref, startv_ref,
          c1w1a_ref, c1w1b_ref, c1b1_ref, c1w2_ref, c1b2_ref,
          c2w1a_ref, c2w1b_ref, c2b1_ref, c2w2_ref, c2b2_ref,
          c3w1a_ref, c3w1b_ref, c3b1_ref, c3w2_ref, c3b2_ref,
          linw_ref, linb_ref,
          out_ref,
          pts_s, gidx_s, sem):
    g = pl.program_id(0)
    s = starts_ref[g]
    e = ends_ref[g]
    astart = (s // _C) * _C
    n_iter = (e - astart + _C - 1) // _C
    startv = startv_ref[0:1, :]
    iota1 = lax.broadcasted_iota(jnp.int32, (_C, _V1), 1)

    def chunk(j):
        off = astart + j * _C
        cp_p = pltpu.make_async_copy(pts_ref.at[pl.ds(off, _C), :], pts_s, sem)
        cp_g = pltpu.make_async_copy(gidx_ref.at[pl.ds(off, _C), :], gidx_s, sem)
        cp_p.start()
        cp_g.start()
        cp_p.wait()
        cp_g.wait()
        p = pts_s[:, 0:3]
        gi = gidx_s[...]
        vox = jnp.floor((p - startv) * 4.0).astype(jnp.int32)
        k = (vox[:, 0:1] * 8 + vox[:, 1:2]) * 8 + vox[:, 2:3]
        key = jnp.where(gi == g, k, -1)
        oh = (key == iota1).astype(jnp.float32)
        return p, oh

    def pass_a(j, acc):
        psum, cnt = acc
        p, oh = chunk(j)
        psum = psum + _dot00(oh, p)
        cnt = cnt + _dot00(oh, jnp.ones((_C, 1), jnp.float32))
        return psum, cnt

    psum0 = jnp.zeros((_V1, 3), jnp.float32)
    cnt0 = jnp.zeros((_V1, 1), jnp.float32)
    psum, count1 = lax.fori_loop(0, n_iter, pass_a, (psum0, cnt0))
    pos1 = psum / jnp.maximum(count1, 1.0)

    def pass_b(j, agg):
        p, oh = chunk(j)
        r = pts_s[:, 3:4]
        rel = p - lax.dot_general(oh, pos1, (((1,), (0,)), ((), ())),
                                  preferred_element_type=jnp.float32, precision=lax.Precision.HIGHEST)
        h = jnp.maximum(
            jnp.dot(r, c1w1a_ref[...], preferred_element_type=jnp.float32, precision=lax.Precision.HIGHEST)
            + jnp.dot(rel, c1w1b_ref[...], preferred_element_type=jnp.float32, precision=lax.Precision.HIGHEST)
            + c1b1_ref[...], 0.0)
        rows = [jnp.max(oh * h[:, f:f + 1], axis=0, keepdims=True)
                for f in range(16)]
        return jnp.maximum(agg, jnp.concatenate(rows, axis=0))

    agg1 = lax.fori_loop(0, n_iter, pass_b, jnp.zeros((16, _V1), jnp.float32))
    f1 = _celu(_dot00(agg1, c1w2_ref[...]) + c1b2_ref[...])

    vox1 = jnp.floor((pos1 - startv) * 2.0).astype(jnp.int32)
    k1 = (vox1[:, 0:1] * 4 + vox1[:, 1:2]) * 4 + vox1[:, 2:3]
    key1 = jnp.where(count1 > 0, k1, -1)
    iota2 = lax.broadcasted_iota(jnp.int32, (_V1, _V2), 1)
    oh2 = (key1 == iota2).astype(jnp.float32)
    psum2 = _dot00(oh2, pos1)
    count2 = _dot00(oh2, jnp.ones((_V1, 1), jnp.float32))
    pos2 = psum2 / jnp.maximum(count2, 1.0)
    rel2 = pos1 - lax.dot_general(oh2, pos2, (((1,), (0,)), ((), ())),
                                  preferred_element_type=jnp.float32, precision=lax.Precision.HIGHEST)
    h2 = jnp.maximum(
        jnp.dot(f1, c2w1a_ref[...], preferred_element_type=jnp.float32, precision=lax.Precision.HIGHEST)
        + jnp.dot(rel2, c2w1b_ref[...], preferred_element_type=jnp.float32, precision=lax.Precision.HIGHEST)
        + c2b1_ref[...], 0.0)
    rows2 = [jnp.max(oh2 * h2[:, f:f + 1], axis=0, keepdims=True)
             for f in range(64)]
    agg2 = jnp.concatenate(rows2, axis=0)
    f2 = _celu(_dot00(agg2, c2w2_ref[...]) + c2b2_ref[...])

    h3 = jnp.maximum(
        jnp.dot(f2, c3w1a_ref[...], preferred_element_type=jnp.float32, precision=lax.Precision.HIGHEST)
        + jnp.dot(pos2, c3w1b_ref[...], preferred_element_type=jnp.float32, precision=lax.Precision.HIGHEST)
        + c3b1_ref[...], 0.0)
    h3 = h3 * (count2 > 0).astype(jnp.float32)
    agg3 = jnp.max(h3, axis=0, keepdims=True)
    f3 = _celu(jnp.dot(agg3, c3w2_ref[...], preferred_element_type=jnp.float32, precision=lax.Precision.HIGHEST)
               + c3b2_ref[...])
    res = (jnp.dot(f3, linw_ref[...], preferred_element_type=jnp.float32, precision=lax.Precision.HIGHEST)
           + linb_ref[...])
    out_ref[...] = res.reshape(1, 1, 256)


def kernel(rgb, pos, glimpse_member__glimpse_index, glimpse__center,
           glimpse__batch,
           c1_w1, c1_b1, c1_w2, c1_b2,
           c2_w1, c2_b1, c2_w2, c2_b2,
           c3_w1, c3_b1, c3_w2, c3_b2,
           lin_w, lin_b):
    del glimpse__batch
    num_g = glimpse__center.shape[0]
    gidx = glimpse_member__glimpse_index.astype(jnp.int32)
    noise = jax.random.uniform(jax.random.key(1), (3,), jnp.float32)
    startv = (jnp.min(pos, axis=0) - noise).reshape(1, 3)
    gr = jnp.arange(num_g, dtype=jnp.int32)
    starts = jnp.searchsorted(gidx, gr, side="left").astype(jnp.int32)
    ends = jnp.searchsorted(gidx, gr + 1, side="left").astype(jnp.int32)
    pts_p = jnp.pad(jnp.concatenate([pos, rgb], axis=1), ((0, _C), (0, 0)))
    gidx_p = jnp.pad(gidx.reshape(-1, 1), ((0, _C), (0, 0)),
                     constant_values=num_g)

    full = lambda a: pl.BlockSpec(a.shape, lambda g, s_, e_: (0,) * a.ndim)
    hbm = pl.BlockSpec(memory_space=pl.ANY)
    operands = [pts_p, gidx_p, startv,
                c1_w1[:1], c1_w1[1:], c1_b1.reshape(1, -1), c1_w2,
                c1_b2.reshape(1, -1),
                c2_w1[:32], c2_w1[32:], c2_b1.reshape(1, -1), c2_w2,
                c2_b2.reshape(1, -1),
                c3_w1[:128], c3_w1[128:], c3_b1.reshape(1, -1), c3_w2,
                c3_b2.reshape(1, -1),
                lin_w, lin_b.reshape(1, -1)]
    grid_spec = pltpu.PrefetchScalarGridSpec(
        num_scalar_prefetch=2,
        grid=(num_g,),
        in_specs=[hbm, hbm] + [full(a) for a in operands[2:]],
        out_specs=pl.BlockSpec((1, 1, 256), lambda g, s_, e_: (g, 0, 0)),
        scratch_shapes=[
            pltpu.VMEM((_C, 4), jnp.float32),
            pltpu.VMEM((_C, 1), jnp.int32),
            pltpu.SemaphoreType.DMA,
        ],
    )
    out = pl.pallas_call(
        _body,
        grid_spec=grid_spec,
        compiler_params=pltpu.CompilerParams(
            dimension_semantics=("parallel",)),
        out_shape=jax.ShapeDtypeStruct((num_g, 1, 256), jnp.float32),
    )(starts, ends, *operands)
    return out.reshape(num_g, 256)


# speedup vs baseline: 1.3037x; 1.0900x over previous
import jax
import jax.numpy as jnp
from jax import lax
from jax.experimental import pallas as pl
from jax.experimental.pallas import tpu as pltpu

_C = 256
_V1 = 512
_V2 = 64


def _celu(x):
    return jnp.where(x > 0, x, jnp.exp(jnp.minimum(x, 0.0)) - 1.0)


def _dot00(a, b):
    return lax.dot_general(a, b, (((0,), (0,)), ((), ())),
                           preferred_element_type=jnp.float32, precision=lax.Precision.HIGHEST)


def _body(starts_ref, ends_ref,
          pts_ref, gidx_ref, startv_ref,
          c1w1a_ref, c1w1b_ref, c1b1_ref, c1w2_ref, c1b2_ref,
          c2w1a_ref, c2w1b_ref, c2b1_ref, c2w2_ref, c2b2_ref,
          c3w1a_ref, c3w1b_ref, c3b1_ref, c3w2_ref, c3b2_ref,
          linw_ref, linb_ref,
          out_ref,
          pts_s, gidx_s, sem):
    g = pl.program_id(0)
    s = starts_ref[g]
    e = ends_ref[g]
    astart = (s // _C) * _C
    n_iter = (e - astart + _C - 1) // _C
    startv = startv_ref[0:1, :]
    iota1 = lax.broadcasted_iota(jnp.int32, (_C, _V1), 1)

    def chunk(j):
        off = astart + j * _C
        cp_p = pltpu.make_async_copy(pts_ref.at[pl.ds(off, _C), :], pts_s, sem)
        cp_g = pltpu.make_async_copy(gidx_ref.at[pl.ds(off, _C), :], gidx_s, sem)
        cp_p.start()
        cp_g.start()
        cp_p.wait()
        cp_g.wait()
        p = pts_s[:, 0:3]
        gi = gidx_s[...]
        vox = jnp.floor((p - startv) * 4.0).astype(jnp.int32)
        k = (vox[:, 0:1] * 8 + vox[:, 1:2]) * 8 + vox[:, 2:3]
        key = jnp.where(gi == g, k, -1)
        oh = (key == iota1).astype(jnp.float32)
        return p, oh

    def pass_a(j, acc):
        psum, cnt = acc
        p, oh = chunk(j)
        psum = psum + _dot00(oh, p)
        cnt = cnt + _dot00(oh, jnp.ones((_C, 1), jnp.float32))
        return psum, cnt

    psum0 = jnp.zeros((_V1, 3), jnp.float32)
    cnt0 = jnp.zeros((_V1, 1), jnp.float32)
    psum, count1 = lax.fori_loop(0, n_iter, pass_a, (psum0, cnt0))
    pos1 = psum / jnp.maximum(count1, 1.0)

    def pass_b(j, agg):
        p, oh = chunk(j)
        r = pts_s[:, 3:4]
        rel = p - lax.dot_general(oh, pos1, (((1,), (0,)), ((), ())),
                                  preferred_element_type=jnp.float32, precision=lax.Precision.HIGHEST)
        h = jnp.maximum(
            jnp.dot(r, c1w1a_ref[...], preferred_element_type=jnp.float32, precision=lax.Precision.HIGHEST)
            + jnp.dot(rel, c1w1b_ref[...], preferred_element_type=jnp.float32, precision=lax.Precision.HIGHEST)
            + c1b1_ref[...], 0.0)
        rows = [jnp.max(oh * h[:, f:f + 1], axis=0, keepdims=True)
                for f in range(16)]
        return jnp.maximum(agg, jnp.concatenate(rows, axis=0))

    agg1 = lax.fori_loop(0, n_iter, pass_b, jnp.zeros((16, _V1), jnp.float32))
    f1 = _celu(_dot00(agg1, c1w2_ref[...]) + c1b2_ref[...])

    vox1 = jnp.floor((pos1 - startv) * 2.0).astype(jnp.int32)
    k1 = (vox1[:, 0:1] * 4 + vox1[:, 1:2]) * 4 + vox1[:, 2:3]
    key1 = jnp.where(count1 > 0, k1, -1)
    iota2 = lax.broadcasted_iota(jnp.int32, (_V1, _V2), 1)
    oh2 = (key1 == iota2).astype(jnp.float32)
    psum2 = _dot00(oh2, pos1)
    count2 = _dot00(oh2, jnp.ones((_V1, 1), jnp.float32))
    pos2 = psum2 / jnp.maximum(count2, 1.0)
    rel2 = pos1 - lax.dot_general(oh2, pos2, (((1,), (0,)), ((), ())),
                                  preferred_element_type=jnp.float32, precision=lax.Precision.HIGHEST)
    h2 = jnp.maximum(
        jnp.dot(f1, c2w1a_ref[...], preferred_element_type=jnp.float32, precision=lax.Precision.HIGHEST)
        + jnp.dot(rel2, c2w1b_ref[...], preferred_element_type=jnp.float32, precision=lax.Precision.HIGHEST)
        + c2b1_ref[...], 0.0)
    rows2 = [jnp.max(oh2 * h2[:, f:f + 1], axis=0, keepdims=True)
             for f in range(64)]
    agg2 = jnp.concatenate(rows2, axis=0)
    f2 = _celu(_dot00(agg2, c2w2_ref[...]) + c2b2_ref[...])

    h3 = jnp.maximum(
        jnp.dot(f2, c3w1a_ref[...], preferred_element_type=jnp.float32, precision=lax.Precision.HIGHEST)
        + jnp.dot(pos2, c3w1b_ref[...], preferred_element_type=jnp.float32, precision=lax.Precision.HIGHEST)
        + c3b1_ref[...], 0.0)
    h3 = h3 * (count2 > 0).astype(jnp.float32)
    agg3 = jnp.max(h3, axis=0, keepdims=True)
    f3 = _celu(jnp.dot(agg3, c3w2_ref[...], preferred_element_type=jnp.float32, precision=lax.Precision.HIGHEST)
               + c3b2_ref[...])
    res = (jnp.dot(f3, linw_ref[...], preferred_element_type=jnp.float32, precision=lax.Precision.HIGHEST)
           + linb_ref[...])
    out_ref[...] = res.reshape(1, 1, 256)


def kernel(rgb, pos, glimpse_member__glimpse_index, glimpse__center,
           glimpse__batch,
           c1_w1, c1_b1, c1_w2, c1_b2,
           c2_w1, c2_b1, c2_w2, c2_b2,
           c3_w1, c3_b1, c3_w2, c3_b2,
           lin_w, lin_b):
    del glimpse__batch
    num_g = glimpse__center.shape[0]
    gidx = glimpse_member__glimpse_index.astype(jnp.int32)
    noise = jax.random.uniform(jax.random.key(1), (3,), jnp.float32)
    startv = (jnp.min(pos, axis=0) - noise).reshape(1, 3)
    gr = jnp.arange(num_g, dtype=jnp.int32)
    starts = jnp.searchsorted(gidx, gr, side="left").astype(jnp.int32)
    ends = jnp.searchsorted(gidx, gr + 1, side="left").astype(jnp.int32)
    pts_p = jnp.pad(jnp.concatenate([pos, rgb], axis=1), ((0, _C), (0, 0)))
    gidx_p = jnp.pad(gidx.reshape(-1, 1), ((0, _C), (0, 0)),
                     constant_values=num_g)

    full = lambda a: pl.BlockSpec(a.shape, lambda g, s_, e_: (0,) * a.ndim)
    hbm = pl.BlockSpec(memory_space=pl.ANY)
    operands = [pts_p, gidx_p, startv,
                c1_w1[:1], c1_w1[1:], c1_b1.reshape(1, -1), c1_w2,
                c1_b2.reshape(1, -1),
                c2_w1[:32], c2_w1[32:], c2_b1.reshape(1, -1), c2_w2,
                c2_b2.reshape(1, -1),
                c3_w1[:128], c3_w1[128:], c3_b1.reshape(1, -1), c3_w2,
                c3_b2.reshape(1, -1),
                lin_w, lin_b.reshape(1, -1)]
    grid_spec = pltpu.PrefetchScalarGridSpec(
        num_scalar_prefetch=2,
        grid=(num_g,),
        in_specs=[hbm, hbm] + [full(a) for a in operands[2:]],
        out_specs=pl.BlockSpec((1, 1, 256), lambda g, s_, e_: (g, 0, 0)),
        scratch_shapes=[
            pltpu.VMEM((_C, 4), jnp.float32),
            pltpu.VMEM((_C, 1), jnp.int32),
            pltpu.SemaphoreType.DMA,
        ],
    )
    out = pl.pallas_call(
        _body,
        grid_spec=grid_spec,
        compiler_params=pltpu.CompilerParams(
            dimension_semantics=("parallel",)),
        out_shape=jax.ShapeDtypeStruct((num_g, 1, 256), jnp.float32),
    )(starts, ends, *operands)
    return out.reshape(num_g, 256)
